# baseline (speedup 1.0000x reference)
import jax
import jax.numpy as jnp
from jax import lax
from jax.experimental import pallas as pl
from jax.experimental.pallas import tpu as pltpu

_C = 80
_H = 128
_W = 128
_K = 10


def _postproc_kernel(off_ref, sz_ref, kp_ref, boxes_ref, cls_ref, sc_ref,
                     scores_ref, pm_ref):
    ninf = jnp.float32(-jnp.inf)
    row = jnp.full((1, _W), ninf, dtype=jnp.float32)
    colv = jnp.full((_H, 1), ninf, dtype=jnp.float32)

    pmax = jnp.zeros((_H, _W), dtype=jnp.float32)
    for c in range(_C):
        xc = kp_ref[c]
        up = jnp.concatenate([xc[1:], row], axis=0)
        dn = jnp.concatenate([row, xc[:-1]], axis=0)
        vy = jnp.maximum(xc, jnp.maximum(up, dn))
        lf = jnp.concatenate([vy[:, 1:], colv], axis=1)
        rt = jnp.concatenate([colv, vy[:, :-1]], axis=1)
        pooled = jnp.maximum(vy, jnp.maximum(lf, rt))
        sc_c = jnp.where(pooled == xc, xc, jnp.float32(0.0))
        scores_ref[c] = sc_c
        pmax = jnp.maximum(pmax, sc_c)

    pm_ref[...] = pmax
    rmax = jnp.max(pmax, axis=1)

    iota_r = lax.iota(jnp.int32, _H)
    lane_row = lax.broadcasted_iota(jnp.int32, (1, _W), 1)
    big = jnp.int32(2**31 - 1)

    wins = []
    win_ys = []
    win_xs = []
    for _ in range(_K):
        m = jnp.max(rmax)
        r = jnp.min(jnp.where(rmax == m, iota_r, big))
        rowv = pm_ref[pl.ds(r, 1), :]
        x_col = jnp.min(jnp.where(rowv == m, lane_row, big))
        newrow = jnp.where(lane_row == x_col, -1.0, rowv)
        pm_ref[pl.ds(r, 1), :] = newrow
        rmax = jnp.where(iota_r == r, jnp.max(newrow), rmax)
        wins.append(r * _W + x_col)
        win_ys.append(r)
        win_xs.append(x_col)

    slabs = []
    decs = []
    masks = []
    for k in range(_K):
        yi = win_ys[k]
        xi = win_xs[k]
        slabs.append(scores_ref[:, pl.ds(yi, 1), :].reshape(1, _C, _W))
        decs.append(jnp.concatenate([
            off_ref[0, pl.ds(yi, 1), :],
            off_ref[1, pl.ds(yi, 1), :],
            sz_ref[0, pl.ds(yi, 1), :],
            sz_ref[1, pl.ds(yi, 1), :],
        ], axis=0).reshape(1, 4, _W))
        masks.append((lane_row == xi).reshape(1, 1, _W))

    mask3 = jnp.concatenate(masks, axis=0)
    slab3 = jnp.concatenate(slabs, axis=0)
    dec3 = jnp.concatenate(decs, axis=0)
    cand = jnp.sum(jnp.where(mask3, slab3, 0.0), axis=2)
    dec = jnp.sum(jnp.where(mask3, dec3, 0.0), axis=2)

    winv = jnp.stack(wins)
    cidx = (winv[:, None] * _C +
            lax.broadcasted_iota(jnp.int32, (_K, _C), 1))

    vals = []
    clss = []
    rows = []
    for _ in range(_K):
        m = jnp.max(cand)
        idx = jnp.min(jnp.where(cand == m, cidx, big))
        cand = jnp.where(cidx == idx, -1.0, cand)

        sp = idx // _C
        cls = idx - sp * _C
        yi = sp // _W
        xi = sp - yi * _W
        y_f = yi.astype(jnp.float32)
        x_f = xi.astype(jnp.float32)

        psel = winv == sp
        o0 = jnp.sum(jnp.where(psel, dec[:, 0], 0.0))
        o1 = jnp.sum(jnp.where(psel, dec[:, 1], 0.0))
        s0 = jnp.sum(jnp.where(psel, dec[:, 2], 0.0))
        s1 = jnp.sum(jnp.where(psel, dec[:, 3], 0.0))

        pos0 = y_f + o1
        pos1 = x_f + o0
        hw0 = s1 * 0.5
        hw1 = s0 * 0.5
        lim = jnp.float32(_W - 1)
        b0 = jnp.clip(pos0 - hw0, 0.0, lim) * 4.0
        b1 = jnp.clip(pos1 - hw1, 0.0, lim) * 4.0
        b2 = jnp.clip(pos0 + hw0, 0.0, lim) * 4.0
        b3 = jnp.clip(pos1 + hw1, 0.0, lim) * 4.0

        vals.append(m)
        clss.append(cls)
        rows.append(jnp.stack([b0, b1, b2, b3]))

    boxes_ref[...] = jnp.stack(rows)
    cls_ref[...] = jnp.stack(clss)
    sc_ref[...] = jnp.stack(vals)


@jax.jit
def kernel(offset, size, keypoint):
    off = offset[0]
    sz = size[0]
    kp = keypoint[0]
    boxes, cls, sc = pl.pallas_call(
        _postproc_kernel,
        out_shape=(
            jax.ShapeDtypeStruct((_K, 4), jnp.float32),
            jax.ShapeDtypeStruct((_K,), jnp.int32),
            jax.ShapeDtypeStruct((_K,), jnp.float32),
        ),
        scratch_shapes=[
            pltpu.VMEM((_C, _H, _W), jnp.float32),
            pltpu.VMEM((_H, _W), jnp.float32),
        ],
    )(off, sz, kp)
    return boxes, cls, sc
